# baseline (speedup 1.0000x reference)
import jax
import jax.numpy as jnp
from jax.experimental import pallas as pl
from jax.experimental.pallas import tpu as pltpu


def _copy_kernel(in_ref, out_ref, sem):
    pltpu.make_async_copy(in_ref, out_ref, sem).start()
    pltpu.make_async_copy(in_ref, out_ref, sem).wait()


def kernel(inputs_embeds, embed_table):
    del embed_table
    return pl.pallas_call(
        _copy_kernel,
        in_specs=[pl.BlockSpec(memory_space=pl.ANY)],
        out_specs=pl.BlockSpec(memory_space=pl.ANY),
        out_shape=jax.ShapeDtypeStruct(inputs_embeds.shape, inputs_embeds.dtype),
        scratch_shapes=[pltpu.SemaphoreType.DMA],
    )(inputs_embeds)


# speedup vs baseline: 12.1817x; 12.1817x over previous
import jax
import jax.numpy as jnp
from jax.experimental import pallas as pl
from jax.experimental.pallas import tpu as pltpu


def _copy_kernel(in_ref, out_ref):
    out_ref[...] = in_ref[...]


def kernel(inputs_embeds, embed_table):
    del embed_table
    b, s, h = inputs_embeds.shape
    x = inputs_embeds.reshape(b * s, h)
    grid = 8
    rows = (b * s) // grid
    out = pl.pallas_call(
        _copy_kernel,
        grid=(grid,),
        in_specs=[pl.BlockSpec((rows, h), lambda i: (i, 0))],
        out_specs=pl.BlockSpec((rows, h), lambda i: (i, 0)),
        out_shape=jax.ShapeDtypeStruct((b * s, h), inputs_embeds.dtype),
    )(x)
    return out.reshape(b, s, h)
